# Initial kernel scaffold; baseline (speedup 1.0000x reference)
import jax
import jax.numpy as jnp
from jax.experimental import pallas as pl


def kernel(x, pitch_emb, velocity_emb, onset_W, onset_b, dur_W, dur_b, proj_W, proj_b):
    raise NotImplementedError("write your pallas kernel here")


# speedup vs baseline: 3.6158x; 3.6158x over previous
import functools

import jax
import jax.numpy as jnp
from jax import lax
from jax.experimental import pallas as pl
from jax.experimental.pallas import tpu as pltpu
from jax.experimental.pallas import tpu_sc as plsc

_B, _L = 4096, 200
_N = _B * _L
_E = 64
_D = 128
_V = 128
_NW = 32
_TPW = _N // _NW
_C = 256
_NCH = _TPW // _C


def _fuse_body(pe, ve, ow, ob, dw, db, pw, pb, tp_out, tv_out, aux_out):
    wp = pw[0:_E, :]
    wo = pw[_E:2 * _E, :]
    wd = pw[2 * _E:3 * _E, :]
    wv = pw[3 * _E:4 * _E, :]
    hi = jax.lax.Precision.HIGHEST
    owv = jnp.dot(ow[:], wo, precision=hi)
    dwv = jnp.dot(dw[:], wd, precision=hi)
    cvec = pb[:] + jnp.dot(ob[:], wo, precision=hi) + jnp.dot(db[:], wd, precision=hi)
    tp_out[:] = jnp.dot(pe[:], wp, precision=hi) + cvec
    tv_out[:] = jnp.dot(ve[:], wv, precision=hi)
    aux_out[:] = jnp.concatenate(
        [owv, dwv, jnp.zeros((6, _D), jnp.float32)], axis=0)


_fuse = pl.pallas_call(
    _fuse_body,
    out_shape=[
        jax.ShapeDtypeStruct((_V, _D), jnp.float32),
        jax.ShapeDtypeStruct((_V, _D), jnp.float32),
        jax.ShapeDtypeStruct((8, _D), jnp.float32),
    ],
)

_mesh = plsc.VectorSubcoreMesh(core_axis_name="c", subcore_axis_name="s")


@functools.partial(
    pl.kernel,
    out_type=jax.ShapeDtypeStruct((_N, _D), jnp.float32),
    mesh=_mesh,
    scratch_types=[
        pltpu.VMEM((_V, _D), jnp.float32),
        pltpu.VMEM((_V, _D), jnp.float32),
        pltpu.VMEM((8, _D), jnp.float32),
        pltpu.VMEM((2, _C), jnp.int32),
        pltpu.VMEM((2, _C), jnp.int32),
        pltpu.VMEM((2, _C), jnp.float32),
        pltpu.VMEM((2, _C), jnp.float32),
        pltpu.VMEM((2, _C, _D), jnp.float32),
        pltpu.SemaphoreType.DMA,
        pltpu.SemaphoreType.DMA,
    ],
)
def _sc_embed(tp_hbm, tv_hbm, aux_hbm, pi_hbm, vi_hbm, on_hbm, du_hbm, out_hbm,
              tp_v, tv_v, aux_v, pxb, vxb, oxb, dxb, obuf, sem0, sem1):
    wid = lax.axis_index("s") * 2 + lax.axis_index("c")
    base = wid * _TPW
    pltpu.sync_copy(tp_hbm, tp_v)
    pltpu.sync_copy(tv_hbm, tv_v)
    pltpu.sync_copy(aux_hbm, aux_v)
    ow = [aux_v[0, pl.ds(16 * j, 16)] for j in range(8)]
    dw = [aux_v[1, pl.ds(16 * j, 16)] for j in range(8)]
    sems = (sem0, sem1)

    def outer(i, carry):
        for b in range(2):
            g = i * 2 + b
            start = base + g * _C
            pltpu.sync_copy(pi_hbm.at[pl.ds(start, _C)], pxb.at[b])
            pltpu.sync_copy(vi_hbm.at[pl.ds(start, _C)], vxb.at[b])
            pltpu.sync_copy(on_hbm.at[pl.ds(start, _C)], oxb.at[b])
            pltpu.sync_copy(du_hbm.at[pl.ds(start, _C)], dxb.at[b])

            @pl.when(i > 0)
            def _wait():
                pltpu.make_async_copy(
                    obuf.at[b], out_hbm.at[pl.ds(start, _C)], sems[b]).wait()

            def group(q, c):
                t0 = q * 16
                pvec = pxb[b, pl.ds(t0, 16)]
                vvec = vxb[b, pl.ds(t0, 16)]
                ovec = oxb[b, pl.ds(t0, 16)]
                dvec = dxb[b, pl.ds(t0, 16)]
                for k in range(16):
                    p = pvec[k]
                    v = vvec[k]
                    ov = jnp.full((16,), ovec[k], jnp.float32)
                    dv = jnp.full((16,), dvec[k], jnp.float32)
                    for j in range(8):
                        sl = pl.ds(16 * j, 16)
                        obuf[b, t0 + k, sl] = (tp_v[p, sl] + tv_v[v, sl]
                                               + ov * ow[j] + dv * dw[j])
                return c

            lax.fori_loop(0, _C // 16, group, 0)
            pltpu.async_copy(obuf.at[b], out_hbm.at[pl.ds(start, _C)], sems[b])
        return carry

    lax.fori_loop(0, _NCH // 2, outer, 0)
    for b in range(2):
        pltpu.make_async_copy(
            obuf.at[b], out_hbm.at[pl.ds(0, _C)], sems[b]).wait()


def kernel(x, pitch_emb, velocity_emb, onset_W, onset_b, dur_W, dur_b,
           proj_W, proj_b):
    tp, tv, aux = _fuse(
        pitch_emb, velocity_emb,
        onset_W, onset_b.reshape(1, _E),
        dur_W, dur_b.reshape(1, _E),
        proj_W, proj_b.reshape(1, _D))
    pi = x[..., 0].astype(jnp.int32).reshape(_N)
    vi = x[..., 3].astype(jnp.int32).reshape(_N)
    on = x[..., 1].reshape(_N)
    du = x[..., 2].reshape(_N)
    out = _sc_embed(tp, tv, aux, pi, vi, on, du)
    return out.reshape(_B, _L, _D)
